# Initial kernel scaffold; baseline (speedup 1.0000x reference)
import jax
import jax.numpy as jnp
from jax.experimental import pallas as pl


def kernel(x, gate_w, w1, w2, w3, sw1, sw2, sw3):
    raise NotImplementedError("write your pallas kernel here")


# speedup vs baseline: 4.7035x; 4.7035x over previous
import functools

import jax
import jax.numpy as jnp
from jax import lax
from jax.experimental import pallas as pl
from jax.experimental.pallas import tpu as pltpu
from jax.experimental.pallas import tpu_sc as plsc

DIM = 2048
HIDDEN = 1024
E = 16
TOP_K = 2

TM = 256
NT = (2 * 2048 * TOP_K) // TM // 2

SC_NC = 2
SC_NS = 16
SC_NW = SC_NC * SC_NS


_FAST = lax.Precision.DEFAULT


def _swiglu_dots(xv, w1v, w3v, w2v):
    a = lax.dot_general(xv, w1v, (((1,), (1,)), ((), ())),
                        preferred_element_type=jnp.float32, precision=_FAST)
    b = lax.dot_general(xv, w3v, (((1,), (1,)), ((), ())),
                        preferred_element_type=jnp.float32, precision=_FAST)
    h = a * jax.nn.sigmoid(a) * b
    return lax.dot_general(h, w2v, (((1,), (1,)), ((), ())),
                           preferred_element_type=jnp.float32, precision=_FAST)


def _shared_body(x_ref, w1_ref, w3_ref, w2_ref, out_ref):
    out_ref[...] = _swiglu_dots(x_ref[...], w1_ref[...], w3_ref[...],
                                w2_ref[...])


def _shared_ffn(x2d, sw1, sw3, sw2):
    n = x2d.shape[0]
    grid = (n // TM,)
    return pl.pallas_call(
        _shared_body,
        grid=grid,
        in_specs=[
            pl.BlockSpec((TM, DIM), lambda i: (i, 0)),
            pl.BlockSpec((HIDDEN, DIM), lambda i: (0, 0)),
            pl.BlockSpec((HIDDEN, DIM), lambda i: (0, 0)),
            pl.BlockSpec((DIM, HIDDEN), lambda i: (0, 0)),
        ],
        out_specs=pl.BlockSpec((TM, DIM), lambda i: (i, 0)),
        out_shape=jax.ShapeDtypeStruct((n, DIM), jnp.float32),
        compiler_params=pltpu.CompilerParams(
            dimension_semantics=("arbitrary",)),
    )(x2d, sw1, sw3, sw2)


def _sc_gather_rows(table, idx):
    b = idx.shape[0]
    d = table.shape[1]
    per = b // SC_NW
    ch = 32
    assert per % ch == 0
    mesh = plsc.VectorSubcoreMesh(core_axis_name="c", subcore_axis_name="s")

    @functools.partial(
        pl.kernel, mesh=mesh,
        out_type=jax.ShapeDtypeStruct((b, d), jnp.float32),
        scratch_types=[
            pltpu.VMEM((ch,), jnp.int32),
            pltpu.VMEM((ch, d), jnp.float32),
            pltpu.SemaphoreType.DMA,
        ],
    )
    def k(table_hbm, idx_hbm, out_hbm, idx_v, rows_v, sem):
        wid = lax.axis_index("s") * SC_NC + lax.axis_index("c")
        base = wid * per
        for c in range(per // ch):
            off = base + c * ch
            pltpu.sync_copy(idx_hbm.at[pl.ds(off, ch)], idx_v)
            pltpu.async_copy(table_hbm.at[idx_v], rows_v, sem).wait()
            pltpu.sync_copy(rows_v, out_hbm.at[pl.ds(off, ch)])

    return k(table, idx)


def _grouped_body(tid_ref, gid_ref, gs_ref, ge_ref,
                  x_ref, ts_ref, w1_ref, w3_ref, w2_ref, out_ref):
    i = pl.program_id(0)
    prev = jnp.maximum(i - 1, 0)
    first = jnp.logical_or(i == 0, tid_ref[i] != tid_ref[prev])

    x = x_ref[...] * ts_ref[...]
    a = lax.dot_general(x, w1_ref[0], (((1,), (1,)), ((), ())),
                        preferred_element_type=jnp.float32, precision=_FAST)
    c = lax.dot_general(x, w3_ref[0], (((1,), (1,)), ((), ())),
                        preferred_element_type=jnp.float32, precision=_FAST)
    h = a * jax.nn.sigmoid(a) * c
    rows = tid_ref[i] * TM + lax.broadcasted_iota(jnp.int32, (TM, 1), 0)
    valid = jnp.logical_and(rows >= gs_ref[i], rows < ge_ref[i])
    hm = jnp.where(valid, h, 0.0)
    piece = lax.dot_general(hm, w2_ref[0], (((1,), (1,)), ((), ())),
                            preferred_element_type=jnp.float32, precision=_FAST)

    @pl.when(first)
    def _():
        out_ref[...] = piece

    @pl.when(jnp.logical_not(first))
    def _():
        out_ref[...] = out_ref[...] + piece


def _grouped_ffn(routed, ts_sorted, w1, w3, w2, tile_ids, gids, gstarts, gends,
                 num_work):
    m = routed.shape[0]
    grid_spec = pltpu.PrefetchScalarGridSpec(
        num_scalar_prefetch=4,
        grid=(num_work,),
        in_specs=[
            pl.BlockSpec((TM, DIM), lambda i, t, g, s, e: (t[i], 0)),
            pl.BlockSpec((TM, 1), lambda i, t, g, s, e: (t[i], 0)),
            pl.BlockSpec((1, HIDDEN, DIM), lambda i, t, g, s, e: (g[i], 0, 0)),
            pl.BlockSpec((1, HIDDEN, DIM), lambda i, t, g, s, e: (g[i], 0, 0)),
            pl.BlockSpec((1, DIM, HIDDEN), lambda i, t, g, s, e: (g[i], 0, 0)),
        ],
        out_specs=pl.BlockSpec((TM, DIM), lambda i, t, g, s, e: (t[i], 0)),
    )
    return pl.pallas_call(
        _grouped_body,
        grid_spec=grid_spec,
        out_shape=jax.ShapeDtypeStruct((m, DIM), jnp.float32),
        compiler_params=pltpu.CompilerParams(
            dimension_semantics=("arbitrary",),
            vmem_limit_bytes=63 * 1024 * 1024),
    )(tile_ids, gids, gstarts, gends, routed, ts_sorted, w1, w3, w2)


def _combine_body(uu_ref, sh_ref, out_ref):
    out_ref[...] = (uu_ref[:, :DIM] + uu_ref[:, DIM:] + sh_ref[...])


def _combine(uu, shared2d):
    n = shared2d.shape[0]
    return pl.pallas_call(
        _combine_body,
        grid=(n // TM,),
        in_specs=[
            pl.BlockSpec((TM, 2 * DIM), lambda i: (i, 0)),
            pl.BlockSpec((TM, DIM), lambda i: (i, 0)),
        ],
        out_specs=pl.BlockSpec((TM, DIM), lambda i: (i, 0)),
        out_shape=jax.ShapeDtypeStruct((n, DIM), jnp.float32),
    )(uu, shared2d)


def _routing_metadata(sel, top_scores, n, num_tiles):
    sel_flat = sel.reshape(-1).astype(jnp.int32)
    m = sel_flat.shape[0]
    oh = (sel_flat[:, None] == jnp.arange(E, dtype=jnp.int32)[None, :])
    oh = oh.astype(jnp.int32)
    within = jnp.cumsum(oh, axis=0) - oh
    counts = jnp.sum(oh, axis=0)
    offsets = jnp.cumsum(counts) - counts
    rank = jnp.take_along_axis(within, sel_flat[:, None], axis=1)[:, 0]
    dest = offsets[sel_flat] + rank
    slot_ids = jnp.arange(m, dtype=jnp.int32)
    tok_sorted = jnp.zeros((m,), jnp.int32).at[dest].set(slot_ids // TOP_K)
    ts_sorted = jnp.zeros((m,), jnp.float32).at[dest].set(
        top_scores.reshape(-1))

    num_work = num_tiles + E - 1
    gstart = offsets
    gend = offsets + counts
    first_tile = gstart // TM
    ntiles_g = jnp.where(counts > 0, (gend - 1) // TM - first_tile + 1, 0)
    cum_incl = jnp.cumsum(ntiles_g)
    cum_excl = cum_incl - ntiles_g
    total = cum_incl[-1]
    w = jnp.arange(num_work, dtype=jnp.int32)
    gid = jnp.searchsorted(cum_incl, w, side="right").astype(jnp.int32)
    gid = jnp.minimum(gid, E - 1)
    tile_id = first_tile[gid] + (w - cum_excl[gid])
    valid_w = w < total
    tile_id = jnp.where(valid_w, tile_id, num_tiles - 1).astype(jnp.int32)
    gs_w = jnp.where(valid_w, gstart[gid], 0).astype(jnp.int32)
    ge_w = jnp.where(valid_w, gend[gid], 0).astype(jnp.int32)
    return tok_sorted, ts_sorted, dest, tile_id, gid, gs_w, ge_w, num_work


def kernel(x, gate_w, w1, w2, w3, sw1, sw2, sw3):
    bs, slen, dim = x.shape
    n = bs * slen
    m = n * TOP_K
    num_tiles = m // TM
    xt = x.reshape(n, dim)

    shared = _shared_ffn(xt, sw1, sw3, sw2)

    scores = jax.nn.sigmoid(xt @ gate_w.T)
    top_scores, sel = lax.top_k(scores, TOP_K)

    (tok_sorted, ts_sorted, dest, tile_id, gid, gs_w, ge_w,
     num_work) = _routing_metadata(sel, top_scores, n, num_tiles)

    routed = _sc_gather_rows(xt, tok_sorted)

    ro = _grouped_ffn(routed, ts_sorted.reshape(m, 1), w1, w3, w2,
                      tile_id, gid, gs_w, ge_w, num_work)

    u = _sc_gather_rows(ro, dest)

    uu = u.reshape(n, 2 * DIM)
    out = _combine(uu, shared)
    return out.reshape(bs, slen, dim)
